# Initial kernel scaffold; baseline (speedup 1.0000x reference)
import jax
import jax.numpy as jnp
from jax.experimental import pallas as pl


def kernel(x, target):
    raise NotImplementedError("write your pallas kernel here")


# speedup vs baseline: 3.3657x; 3.3657x over previous
import functools
import math

import jax
import jax.numpy as jnp
from jax.experimental import pallas as pl
from jax.experimental.pallas import tpu as pltpu

_SIZE = 32000
_PAD = 0
_SMOOTH = 0.1
_CONF = 1.0 - _SMOOTH
_EPS = _SMOOTH / (_SIZE - 1)
_C0 = (_SIZE - 1) * _EPS * math.log(_EPS) + _CONF * math.log(_CONF)

_ROWS_BLK = 256
_COLS_BLK = 1280


def _body(tgt_ref, x_ref, out_ref):
    i = pl.program_id(0)
    j = pl.program_id(1)

    @pl.when((i == 0) & (j == 0))
    def _init():
        out_ref[0] = 0.0
        out_ref[1] = 0.0

    x = x_ref[...]
    tgt = tgt_ref[...]
    mask = (tgt != _PAD).astype(jnp.float32)
    col = jax.lax.broadcasted_iota(jnp.int32, x.shape, 1) + j * _COLS_BLK
    match = (col == tgt).astype(jnp.float32)
    w = (_EPS - _CONF) * match - _EPS
    out_ref[0] += jnp.sum(x * w * mask)

    @pl.when(j == 0)
    def _tok():
        out_ref[1] += jnp.sum(mask)


@jax.jit
def kernel(x, target):
    n = x.shape[0]
    tgt2d = target.astype(jnp.int32).reshape(n, 1)
    grid = (n // _ROWS_BLK, _SIZE // _COLS_BLK)
    out = pl.pallas_call(
        _body,
        grid=grid,
        in_specs=[
            pl.BlockSpec((_ROWS_BLK, 1), lambda i, j: (i, 0)),
            pl.BlockSpec((_ROWS_BLK, _COLS_BLK), lambda i, j: (i, j)),
        ],
        out_specs=pl.BlockSpec(memory_space=pltpu.SMEM),
        out_shape=jax.ShapeDtypeStruct((2,), jnp.float32),
    )(tgt2d, x)
    acc, tokens = out[0], out[1]
    return (jnp.float32(_C0) * tokens + acc) / tokens
